# Initial kernel scaffold; baseline (speedup 1.0000x reference)
import jax
import jax.numpy as jnp
from jax.experimental import pallas as pl


def kernel(bias, qlen, klen):
    raise NotImplementedError("write your pallas kernel here")


# speedup vs baseline: 42.7388x; 42.7388x over previous
import functools

import jax
import jax.numpy as jnp
from jax import lax
from jax.experimental import pallas as pl
from jax.experimental.pallas import tpu as pltpu
from jax.experimental.pallas import tpu_sc as plsc

H = 16
TAB = 257
SEQ = 2048
NSHIFT = 16
GW = 4080
PAD = 1919


def _build_g16(bias):
    def body(bias_ref, g16_ref):
        b = bias_ref[...]
        left = jnp.broadcast_to(b[:, :1], (H, PAD))
        right = jnp.broadcast_to(b[:, TAB - 1 : TAB], (H, PAD))
        g = jnp.concatenate([left, b, right], axis=1)
        for a in range(NSHIFT):
            g16_ref[:, a, :] = g[:, 15 - a : 15 - a + GW]

    return pl.pallas_call(
        body,
        out_shape=jax.ShapeDtypeStruct((H, NSHIFT, GW), jnp.float32),
    )(bias)


def _materialize(g16):
    info = plsc.get_sparse_core_info()
    nc, ns = info.num_cores, info.num_subcores
    groups_per_worker = (H * SEQ) // (nc * ns) // NSHIFT
    half = SEQ // nc

    mesh = plsc.VectorSubcoreMesh(core_axis_name="c", subcore_axis_name="s")

    @functools.partial(
        pl.kernel,
        mesh=mesh,
        out_type=jax.ShapeDtypeStruct((H, SEQ, SEQ), jnp.float32),
        scratch_types=[pltpu.VMEM((NSHIFT, GW), jnp.float32)],
        compiler_params=pltpu.CompilerParams(use_tc_tiling_on_sc=False),
    )
    def body(g16_hbm, out_hbm, g_vmem):
        h = lax.axis_index("s")
        c = lax.axis_index("c")
        pltpu.sync_copy(g16_hbm.at[h], g_vmem)

        def step(gi, carry):
            i0 = c * half + gi * NSHIFT
            off = pl.multiple_of(2032 - i0, NSHIFT)
            pltpu.sync_copy(
                g_vmem.at[:, pl.ds(off, SEQ)],
                out_hbm.at[h, pl.ds(i0, NSHIFT), :],
            )
            return carry

        lax.fori_loop(0, groups_per_worker, step, 0)

    return body(g16)


def kernel(bias, qlen, klen):
    del qlen, klen
    return _materialize(_build_g16(bias))


# speedup vs baseline: 42.7743x; 1.0008x over previous
import functools

import jax
import jax.numpy as jnp
from jax import lax
from jax.experimental import pallas as pl
from jax.experimental.pallas import tpu as pltpu
from jax.experimental.pallas import tpu_sc as plsc

H = 16
TAB = 257
SEQ = 2048
NSHIFT = 16
GW = 4080
PAD = 1919


def _build_g16(bias):
    def body(bias_ref, g16_ref):
        b = bias_ref[...]
        left = jnp.broadcast_to(b[:, :1], (H, PAD))
        right = jnp.broadcast_to(b[:, TAB - 1 : TAB], (H, PAD))
        g = jnp.concatenate([left, b, right], axis=1)
        for a in range(NSHIFT):
            g16_ref[:, a, :] = g[:, 15 - a : 15 - a + GW]

    return pl.pallas_call(
        body,
        out_shape=jax.ShapeDtypeStruct((H, NSHIFT, GW), jnp.float32),
    )(bias)


def _materialize(g16):
    info = plsc.get_sparse_core_info()
    nc, ns = info.num_cores, info.num_subcores
    groups_per_worker = (H * SEQ) // (nc * ns) // NSHIFT
    half = SEQ // nc

    mesh = plsc.VectorSubcoreMesh(core_axis_name="c", subcore_axis_name="s")

    @functools.partial(
        pl.kernel,
        mesh=mesh,
        out_type=jax.ShapeDtypeStruct((H, SEQ, SEQ), jnp.float32),
        scratch_types=[
            pltpu.VMEM((NSHIFT, GW), jnp.float32),
            pltpu.SemaphoreType.DMA,
        ],
        compiler_params=pltpu.CompilerParams(use_tc_tiling_on_sc=False),
    )
    def body(g16_hbm, out_hbm, g_vmem, sem):
        h = lax.axis_index("s")
        c = lax.axis_index("c")
        pltpu.sync_copy(g16_hbm.at[h], g_vmem)

        lag = 4

        def group_copy(gi):
            i0 = c * half + gi * NSHIFT
            off = pl.multiple_of(2032 - i0, NSHIFT)
            return pltpu.make_async_copy(
                g_vmem.at[:, pl.ds(off, SEQ)],
                out_hbm.at[h, pl.ds(i0, NSHIFT), :],
                sem,
            )

        def fire(gi, carry):
            group_copy(gi).start()

            @pl.when(gi >= lag)
            def _():
                group_copy(gi - lag).wait()

            return carry

        lax.fori_loop(0, groups_per_worker, fire, 0)

        def drain(gi, carry):
            group_copy(gi).wait()
            return carry

        lax.fori_loop(groups_per_worker - lag, groups_per_worker, drain, 0)

    return body(g16)


def kernel(bias, qlen, klen):
    del qlen, klen
    return _materialize(_build_g16(bias))


# speedup vs baseline: 130.0920x; 3.0414x over previous
import functools

import jax
import jax.numpy as jnp
from jax import lax
from jax.experimental import pallas as pl
from jax.experimental.pallas import tpu as pltpu
from jax.experimental.pallas import tpu_sc as plsc

H = 16
TAB = 257
SEQ = 2048
PAD = 1919
SU = 4088
NQ = 31
NCLS = 16
NK = 16


def _build_s8(bias):
    def body(bias_ref, s8_ref):
        b = bias_ref[...]
        left = jnp.broadcast_to(b[:, :1], (H, PAD))
        right = jnp.broadcast_to(b[:, TAB - 1 : TAB], (H, PAD))
        g = jnp.concatenate([left, b, right], axis=1)
        for sh in range(8):
            s8_ref[:, sh, :] = g[:, 7 - sh : 7 - sh + SU]

    return pl.pallas_call(
        body,
        out_shape=jax.ShapeDtypeStruct((H, 8, SU), jnp.float32),
    )(bias)


def _materialize(s8):
    mesh = plsc.VectorSubcoreMesh(core_axis_name="c", subcore_axis_name="s")

    @functools.partial(
        pl.kernel,
        mesh=mesh,
        out_type=jax.ShapeDtypeStruct((H, SEQ // 8, SEQ // 128, 8, 128), jnp.float32),
        scratch_types=[
            pltpu.VMEM((2, NQ, 8, 128), jnp.float32),
            pltpu.SemaphoreType.DMA,
            pltpu.SemaphoreType.DMA,
        ],
        compiler_params=pltpu.CompilerParams(use_tc_tiling_on_sc=False),
    )
    def body(s8_hbm, out_hbm, l_vmem, sem_stage, sem_out):
        h = lax.axis_index("s")
        c = lax.axis_index("c")

        def stage_one(cls, q):
            m = c * 8 + cls
            r0 = pl.multiple_of(120 - 8 * m, 8)
            return pltpu.make_async_copy(
                s8_hbm.at[h, :, pl.ds(128 * q + r0, 128)],
                l_vmem.at[cls % 2, q],
                sem_stage,
            )

        def stage_start(cls):
            def go(q, carry):
                stage_one(cls, q).start()
                return carry

            lax.fori_loop(0, NQ, go, 0)

        def stage_wait(cls):
            def go(q, carry):
                stage_one(cls, q).wait()
                return carry

            lax.fori_loop(0, NQ, go, 0)

        def out_one(cls, k):
            m = c * 8 + cls
            return pltpu.make_async_copy(
                l_vmem.at[cls % 2, pl.ds(15 - k, NK)],
                out_hbm.at[h, m + NCLS * k],
                sem_out,
            )

        def out_start(cls):
            def go(k, carry):
                out_one(cls, k).start()
                return carry

            lax.fori_loop(0, NK, go, 0)

        def out_wait(cls):
            def go(k, carry):
                out_one(cls, k).wait()
                return carry

            lax.fori_loop(0, NK, go, 0)

        stage_start(0)
        for cls in range(8):
            stage_wait(cls)
            out_start(cls)
            if cls < 7:
                stage_start(cls + 1)
            out_wait(cls)

    return body(s8)


def kernel(bias, qlen, klen):
    del qlen, klen
    out5 = _materialize(_build_s8(bias))
    return out5.transpose(0, 1, 3, 2, 4).reshape(H, SEQ, SEQ)


# speedup vs baseline: 134.7503x; 1.0358x over previous
import functools

import jax
import jax.numpy as jnp
from jax import lax
from jax.experimental import pallas as pl
from jax.experimental.pallas import tpu as pltpu
from jax.experimental.pallas import tpu_sc as plsc

H = 16
TAB = 257
SEQ = 2048
PAD = 1919
SU = 4088
NQ = 31
NCLS = 16
NK = 16


def _build_s8(bias):
    def body(bias_ref, s8_ref):
        b = bias_ref[...]
        left = jnp.broadcast_to(b[:, :1], (H, PAD))
        right = jnp.broadcast_to(b[:, TAB - 1 : TAB], (H, PAD))
        g = jnp.concatenate([left, b, right], axis=1)
        for sh in range(8):
            s8_ref[:, sh, :] = g[:, 7 - sh : 7 - sh + SU]

    return pl.pallas_call(
        body,
        out_shape=jax.ShapeDtypeStruct((H, 8, SU), jnp.float32),
    )(bias)


def _materialize(s8):
    mesh = plsc.VectorSubcoreMesh(core_axis_name="c", subcore_axis_name="s")

    @functools.partial(
        pl.kernel,
        mesh=mesh,
        out_type=jax.ShapeDtypeStruct((H, SEQ // 8, SEQ // 128, 8, 128), jnp.float32),
        scratch_types=[
            pltpu.VMEM((2, 2, NQ, 8, 128), jnp.float32),
            pltpu.SemaphoreType.DMA,
            pltpu.SemaphoreType.DMA,
        ],
        compiler_params=pltpu.CompilerParams(use_tc_tiling_on_sc=False),
    )
    def body(s8_hbm, out_hbm, l_vmem, sem_stage, sem_out):
        h = lax.axis_index("s")
        c = lax.axis_index("c")

        def stage_one(pr, pi, q):
            m0 = c * 8 + 2 * pr
            r0 = pl.multiple_of(120 - 8 * m0 - 8 * pi, 8)
            return pltpu.make_async_copy(
                s8_hbm.at[h, :, pl.ds(128 * q + r0, 128)],
                l_vmem.at[pr % 2, pi, q],
                sem_stage,
            )

        def stage_start(pr):
            def go(t, carry):
                stage_one(pr, t // NQ, t % NQ).start()
                return carry

            lax.fori_loop(0, 2 * NQ, go, 0)

        def stage_wait(pr):
            def go(t, carry):
                stage_one(pr, t // NQ, t % NQ).wait()
                return carry

            lax.fori_loop(0, 2 * NQ, go, 0)

        def out_one(pr, k):
            m0 = c * 8 + 2 * pr
            return pltpu.make_async_copy(
                l_vmem.at[pr % 2, :, pl.ds(15 - k, NK)],
                out_hbm.at[h, pl.ds(m0 + NCLS * k, 2)],
                sem_out,
            )

        def out_start(pr):
            def go(k, carry):
                out_one(pr, k).start()
                return carry

            lax.fori_loop(0, NK, go, 0)

        def out_wait(pr):
            def go(k, carry):
                out_one(pr, k).wait()
                return carry

            lax.fori_loop(0, NK, go, 0)

        stage_start(0)
        for pr in range(4):
            stage_wait(pr)
            out_start(pr)
            if pr < 3:
                stage_start(pr + 1)
            out_wait(pr)

    return body(s8)


def kernel(bias, qlen, klen):
    del qlen, klen
    out5 = _materialize(_build_s8(bias))
    return out5.transpose(0, 1, 3, 2, 4).reshape(H, SEQ, SEQ)
